# Initial kernel scaffold; baseline (speedup 1.0000x reference)
import jax
import jax.numpy as jnp
from jax.experimental import pallas as pl


def kernel(x, Wqkv, bqkv, Wproj, bproj):
    raise NotImplementedError("write your pallas kernel here")


# speedup vs baseline: 1.9014x; 1.9014x over previous
import functools

import jax
import jax.numpy as jnp
from jax.experimental import pallas as pl
from jax.experimental.pallas import tpu as pltpu

DIM = 256
NUM_HEADS = 8
N_WIN = (2, 4, 4)
TOPK_N = 4
THRESH = 2.0


def _routing_body(x_ref, w_ref, b_ref, idx_ref, *, nw, tws, scale):
    acc = x_ref[0, 0]
    for t in range(1, x_ref.shape[0]):
        acc = acc + x_ref[t, 0]
    r = jnp.sum(acc, axis=1) * (1.0 / tws)
    wq = w_ref[0:DIM, :]
    wk = w_ref[DIM:2 * DIM, :]
    qr = jax.lax.dot_general(r, wq, (((1,), (1,)), ((), ())),
                             preferred_element_type=jnp.float32)
    kr = jax.lax.dot_general(r, wk, (((1,), (1,)), ((), ())),
                             preferred_element_type=jnp.float32)
    qr = qr + b_ref[:, 0:DIM]
    kr = kr + b_ref[:, DIM:2 * DIM]
    a = jax.lax.dot_general(qr, kr, (((1,), (1,)), ((), ())),
                            preferred_element_type=jnp.float32) * scale
    iota_j = jax.lax.broadcasted_iota(jnp.int32, (nw, nw), 1)
    iota_f = iota_j.astype(jnp.float32)
    for kk in range(TOPK_N):
        m = jnp.max(a, axis=1, keepdims=True)
        cand = jnp.where(a >= m, iota_f, 1e9)
        jmin = jnp.min(cand, axis=1, keepdims=True)
        sel = iota_f == jmin
        idx_ref[0, :, kk:kk + 1] = jmin.astype(jnp.int32)
        a = jnp.where(sel, -1e30, a)


def _main_body(idx_ref, x_ref, w_ref, b_ref, wp_ref, bp_ref, out_ref,
               s_scr, o_scr, *, nw, ws, scale):
    b = pl.program_id(1)
    xv = x_ref[0, 0].reshape(nw * ws, DIM)
    qkv = jax.lax.dot_general(xv, w_ref[...], (((1,), (1,)), ((), ())),
                              preferred_element_type=jnp.float32)
    qkv = qkv + b_ref[...]
    s_scr[...] = (qkv >= THRESH).astype(jnp.float32)
    mask_r = jax.lax.broadcasted_iota(jnp.int32, (DIM, DIM), 0) // (DIM // NUM_HEADS)
    mask_c = jax.lax.broadcasted_iota(jnp.int32, (DIM, DIM), 1) // (DIM // NUM_HEADS)
    mask = (mask_r == mask_c).astype(jnp.float32)
    for i in range(nw):
        acc = jnp.zeros((DIM, DIM), jnp.float32)
        for kk in range(TOPK_N):
            j = idx_ref[b, i, kk]
            ks = s_scr[pl.ds(j * ws, ws), DIM:2 * DIM]
            vs = s_scr[pl.ds(j * ws, ws), 2 * DIM:3 * DIM]
            acc = acc + jax.lax.dot_general(
                ks, vs, (((0,), (0,)), ((), ())),
                preferred_element_type=jnp.float32)
        kvm = acc * mask
        qi = s_scr[i * ws:(i + 1) * ws, 0:DIM]
        oi = jax.lax.dot_general(qi, kvm, (((1,), (0,)), ((), ())),
                                 preferred_element_type=jnp.float32) * scale
        o_scr[i * ws:(i + 1) * ws, :] = oi
    outp = jax.lax.dot_general(o_scr[...], wp_ref[...], (((1,), (1,)), ((), ())),
                               preferred_element_type=jnp.float32)
    outp = outp + bp_ref[...]
    out_ref[0, 0] = outp.reshape(nw, ws, DIM)


def kernel(x, Wqkv, bqkv, Wproj, bproj):
    T, B, Lt, Lh, Lw, C = x.shape
    wt, wh, ww = N_WIN
    nw = wt * wh * ww
    ws = (Lt // wt) * (Lh // wh) * (Lw // ww)
    H = NUM_HEADS
    hd = C // H
    scale = hd ** (-0.5)

    x_win = x.reshape(T, B, wt, Lt // wt, wh, Lh // wh, ww, Lw // ww, C)
    x_win = jnp.transpose(x_win, (0, 1, 2, 4, 6, 3, 5, 7, 8))
    x_win = x_win.reshape(T, B, nw, ws, C)
    b2 = bqkv.reshape(1, 3 * C)
    bp2 = bproj.reshape(1, C)

    idx = pl.pallas_call(
        functools.partial(_routing_body, nw=nw, tws=T * ws, scale=scale),
        grid=(B,),
        in_specs=[
            pl.BlockSpec((T, 1, nw, ws, C), lambda b: (0, b, 0, 0, 0)),
            pl.BlockSpec((3 * C, C), lambda b: (0, 0)),
            pl.BlockSpec((1, 3 * C), lambda b: (0, 0)),
        ],
        out_specs=pl.BlockSpec((1, nw, TOPK_N), lambda b: (b, 0, 0)),
        out_shape=jax.ShapeDtypeStruct((B, nw, TOPK_N), jnp.int32),
    )(x_win, Wqkv, b2)

    out_win = pl.pallas_call(
        functools.partial(_main_body, nw=nw, ws=ws, scale=scale),
        grid=(T, B),
        in_specs=[
            pl.BlockSpec(memory_space=pltpu.SMEM),
            pl.BlockSpec((1, 1, nw, ws, C), lambda t, b: (t, b, 0, 0, 0)),
            pl.BlockSpec((3 * C, C), lambda t, b: (0, 0)),
            pl.BlockSpec((1, 3 * C), lambda t, b: (0, 0)),
            pl.BlockSpec((C, C), lambda t, b: (0, 0)),
            pl.BlockSpec((1, C), lambda t, b: (0, 0)),
        ],
        out_specs=pl.BlockSpec((1, 1, nw, ws, C), lambda t, b: (t, b, 0, 0, 0)),
        out_shape=jax.ShapeDtypeStruct((T, B, nw, ws, C), jnp.float32),
        scratch_shapes=[
            pltpu.VMEM((nw * ws, 3 * C), jnp.float32),
            pltpu.VMEM((nw * ws, C), jnp.float32),
        ],
    )(idx, x_win, Wqkv, b2, Wproj, bp2)

    out = out_win.reshape(T, B, wt, wh, ww, Lt // wt, Lh // wh, Lw // ww, C)
    out = jnp.transpose(out, (0, 1, 2, 5, 3, 6, 4, 7, 8))
    return out.reshape(T, B, Lt, Lh, Lw, C)
